# baseline (speedup 1.0000x reference)
import functools

import jax
import jax.numpy as jnp
from jax import lax
from jax.experimental import pallas as pl
from jax.experimental.pallas import tpu as pltpu
from jax.experimental.pallas import tpu_sc as plsc

NUM_WORKERS = 32
BATCH = 16384
B_PER_W = BATCH // NUM_WORKERS
N_CHUNKS = 4
CHUNK = B_PER_W // N_CHUNKS
EMBED = 64
LANES = 16


def _sc_body(uid_hbm, iid_hbm, utab_hbm, itab_hbm, out_hbm,
             idx_u, idx_i, u_rows, i_rows, out_v, sem):
    wid = lax.axis_index("s") * 2 + lax.axis_index("c")

    pltpu.sync_copy(uid_hbm.at[wid], idx_u)
    pltpu.sync_copy(iid_hbm.at[wid], idx_i)

    copies = []
    for j in range(N_CHUNKS):
        dst = u_rows.at[pl.ds(j * CHUNK, CHUNK)]
        copies.append(pltpu.async_copy(utab_hbm.at[idx_u.at[j]], dst, sem))
        dst = i_rows.at[pl.ds(j * CHUNK, CHUNK)]
        copies.append(pltpu.async_copy(itab_hbm.at[idx_i.at[j]], dst, sem))
    for c in copies:
        c.wait()

    iota = lax.iota(jnp.int32, LANES)

    def chunk_body(c, _):
        rvec = c * LANES + iota
        acc = jnp.zeros((LANES,), jnp.float32)
        cvec = iota
        for _d in range(EMBED):
            u = plsc.load_gather(u_rows, [rvec, cvec])
            v = plsc.load_gather(i_rows, [rvec, cvec])
            acc = acc + u * v
            cvec = (cvec + 1) & (EMBED - 1)
        out_v[pl.ds(c * LANES, LANES)] = acc
        return 0

    lax.fori_loop(0, B_PER_W // LANES, chunk_body, 0)

    pltpu.sync_copy(out_v, out_hbm.at[pl.ds(wid * B_PER_W, B_PER_W)])


@jax.jit
def kernel(user_ids, item_ids, user_table, item_table):
    uids = user_ids.astype(jnp.int32).reshape(NUM_WORKERS, N_CHUNKS, CHUNK)
    iids = item_ids.astype(jnp.int32).reshape(NUM_WORKERS, N_CHUNKS, CHUNK)
    mesh = plsc.VectorSubcoreMesh(core_axis_name="c", subcore_axis_name="s")
    run = pl.kernel(
        _sc_body,
        out_type=jax.ShapeDtypeStruct((BATCH,), jnp.float32),
        mesh=mesh,
        compiler_params=pltpu.CompilerParams(
            needs_layout_passes=False, use_tc_tiling_on_sc=False
        ),
        scratch_types=[
            pltpu.VMEM((N_CHUNKS, CHUNK), jnp.int32),
            pltpu.VMEM((N_CHUNKS, CHUNK), jnp.int32),
            pltpu.VMEM((B_PER_W, EMBED), jnp.float32),
            pltpu.VMEM((B_PER_W, EMBED), jnp.float32),
            pltpu.VMEM((B_PER_W,), jnp.float32),
            pltpu.SemaphoreType.DMA,
        ],
    )
    return run(uids, iids, user_table, item_table)


# speedup vs baseline: 1.5604x; 1.5604x over previous
import jax
import jax.numpy as jnp
from jax import lax
from jax.experimental import pallas as pl
from jax.experimental.pallas import tpu as pltpu
from jax.experimental.pallas import tpu_sc as plsc

NUM_WORKERS = 32
BATCH = 16384
B_PER_W = BATCH // NUM_WORKERS
N_ROUNDS = 8
ROUND = B_PER_W // N_ROUNDS
EMBED = 64
LANES = 16


def _sc_body(uid_hbm, iid_hbm, utab_hbm, itab_hbm, out_hbm,
             idx_u, idx_i, u_buf, i_buf, out_v, sem):
    wid = lax.axis_index("s") * 2 + lax.axis_index("c")

    pltpu.sync_copy(uid_hbm.at[wid], idx_u)
    pltpu.sync_copy(iid_hbm.at[wid], idx_i)

    iota = lax.iota(jnp.int32, LANES)

    def round_body(r, _):
        base = r * ROUND

        def fire_body(g, _):
            uvec = idx_u[pl.ds(base + g * LANES, LANES)]
            ivec = idx_i[pl.ds(base + g * LANES, LANES)]
            for l in range(LANES):
                j = g * LANES + l
                pltpu.async_copy(utab_hbm.at[uvec[l]], u_buf.at[j], sem)
                pltpu.async_copy(itab_hbm.at[ivec[l]], i_buf.at[j], sem)
            return 0

        lax.fori_loop(0, ROUND // LANES, fire_body, 0)
        pltpu.make_async_copy(utab_hbm.at[pl.ds(0, ROUND)], u_buf, sem).wait()
        pltpu.make_async_copy(itab_hbm.at[pl.ds(0, ROUND)], i_buf, sem).wait()

        def chunk_body(c, _):
            evec = c * LANES + iota
            acc = jnp.zeros((LANES,), jnp.float32)
            cvec = iota
            for _d in range(EMBED):
                u = plsc.load_gather(u_buf, [evec, cvec])
                v = plsc.load_gather(i_buf, [evec, cvec])
                acc = acc + u * v
                cvec = (cvec + 1) & (EMBED - 1)
            out_v[pl.ds(base + c * LANES, LANES)] = acc
            return 0

        lax.fori_loop(0, ROUND // LANES, chunk_body, 0)
        return 0

    lax.fori_loop(0, N_ROUNDS, round_body, 0)

    pltpu.sync_copy(out_v, out_hbm.at[pl.ds(wid * B_PER_W, B_PER_W)])


@jax.jit
def kernel(user_ids, item_ids, user_table, item_table):
    uids = user_ids.astype(jnp.int32).reshape(NUM_WORKERS, B_PER_W)
    iids = item_ids.astype(jnp.int32).reshape(NUM_WORKERS, B_PER_W)
    mesh = plsc.VectorSubcoreMesh(core_axis_name="c", subcore_axis_name="s")
    run = pl.kernel(
        _sc_body,
        out_type=jax.ShapeDtypeStruct((BATCH,), jnp.float32),
        mesh=mesh,
        compiler_params=pltpu.CompilerParams(
            needs_layout_passes=False
        ),
        scratch_types=[
            pltpu.VMEM((B_PER_W,), jnp.int32),
            pltpu.VMEM((B_PER_W,), jnp.int32),
            pltpu.VMEM((ROUND, EMBED), jnp.float32),
            pltpu.VMEM((ROUND, EMBED), jnp.float32),
            pltpu.VMEM((B_PER_W,), jnp.float32),
            pltpu.SemaphoreType.DMA,
        ],
    )
    return run(uids, iids, user_table, item_table)


# speedup vs baseline: 1.5752x; 1.0095x over previous
import jax
import jax.numpy as jnp
from jax import lax
from jax.experimental import pallas as pl
from jax.experimental.pallas import tpu as pltpu
from jax.experimental.pallas import tpu_sc as plsc

NUM_WORKERS = 32
BATCH = 16384
B_PER_W = BATCH // NUM_WORKERS
N_ROUNDS = 2
ROUND = B_PER_W // N_ROUNDS
EMBED = 64
LANES = 16


def _sc_body(uid_hbm, iid_hbm, utab_hbm, itab_hbm, out_hbm,
             idx_u, idx_i, u_buf, i_buf, out_v, sem):
    wid = lax.axis_index("s") * 2 + lax.axis_index("c")

    pltpu.sync_copy(uid_hbm.at[wid], idx_u)
    pltpu.sync_copy(iid_hbm.at[wid], idx_i)

    iota = lax.iota(jnp.int32, LANES)

    def round_body(r, _):
        base = r * ROUND

        def fire_body(g, _):
            uvec = idx_u[pl.ds(base + g * LANES, LANES)]
            ivec = idx_i[pl.ds(base + g * LANES, LANES)]
            for l in range(LANES):
                j = g * LANES + l
                pltpu.async_copy(utab_hbm.at[uvec[l]], u_buf.at[j], sem)
                pltpu.async_copy(itab_hbm.at[ivec[l]], i_buf.at[j], sem)
            return 0

        lax.fori_loop(0, ROUND // LANES, fire_body, 0)
        pltpu.make_async_copy(utab_hbm.at[pl.ds(0, ROUND)], u_buf, sem).wait()
        pltpu.make_async_copy(itab_hbm.at[pl.ds(0, ROUND)], i_buf, sem).wait()

        def chunk_body(c, _):
            evec = c * LANES + iota
            acc = jnp.zeros((LANES,), jnp.float32)
            cvec = iota
            for _d in range(EMBED):
                u = plsc.load_gather(u_buf, [evec, cvec])
                v = plsc.load_gather(i_buf, [evec, cvec])
                acc = acc + u * v
                cvec = (cvec + 1) & (EMBED - 1)
            out_v[pl.ds(base + c * LANES, LANES)] = acc
            return 0

        lax.fori_loop(0, ROUND // LANES, chunk_body, 0)
        return 0

    lax.fori_loop(0, N_ROUNDS, round_body, 0)

    pltpu.sync_copy(out_v, out_hbm.at[pl.ds(wid * B_PER_W, B_PER_W)])


@jax.jit
def kernel(user_ids, item_ids, user_table, item_table):
    uids = user_ids.astype(jnp.int32).reshape(NUM_WORKERS, B_PER_W)
    iids = item_ids.astype(jnp.int32).reshape(NUM_WORKERS, B_PER_W)
    mesh = plsc.VectorSubcoreMesh(core_axis_name="c", subcore_axis_name="s")
    run = pl.kernel(
        _sc_body,
        out_type=jax.ShapeDtypeStruct((BATCH,), jnp.float32),
        mesh=mesh,
        compiler_params=pltpu.CompilerParams(
            needs_layout_passes=False
        ),
        scratch_types=[
            pltpu.VMEM((B_PER_W,), jnp.int32),
            pltpu.VMEM((B_PER_W,), jnp.int32),
            pltpu.VMEM((ROUND, EMBED), jnp.float32),
            pltpu.VMEM((ROUND, EMBED), jnp.float32),
            pltpu.VMEM((B_PER_W,), jnp.float32),
            pltpu.SemaphoreType.DMA,
        ],
    )
    return run(uids, iids, user_table, item_table)
